# baseline (speedup 1.0000x reference)
import functools

import jax
import jax.numpy as jnp
from jax import lax
from jax.experimental import pallas as pl
from jax.experimental.pallas import tpu as pltpu
from jax.experimental.pallas import tpu_sc as plsc


def _broadcast_table(table, B):
    L, F = table.shape
    info = plsc.get_sparse_core_info()
    NC, NS = info.num_cores, info.num_subcores
    NW = NC * NS
    rows_per = L // NW
    assert rows_per * NW == L and (rows_per * F) % 8 == 0

    mesh = plsc.VectorSubcoreMesh(core_axis_name="c", subcore_axis_name="s")

    @functools.partial(
        pl.kernel,
        mesh=mesh,
        out_type=jax.ShapeDtypeStruct((B, L, F), table.dtype),
        scratch_types=[
            pltpu.VMEM((rows_per, F), table.dtype),
            pltpu.SemaphoreType.DMA,
            pltpu.SemaphoreType.DMA,
        ],
    )
    def k(table_hbm, out_hbm, buf, sem_r, sem_w):
        wid = lax.axis_index("s") * NC + lax.axis_index("c")
        base = wid * rows_per
        half = rows_per // 2
        r0 = pltpu.async_copy(
            table_hbm.at[pl.ds(base, half)], buf.at[pl.ds(0, half)], sem_r)
        r1 = pltpu.async_copy(
            table_hbm.at[pl.ds(base + half, half)], buf.at[pl.ds(half, half)],
            sem_r)
        r0.wait()
        writes = []
        for b in range(B):
            writes.append(pltpu.async_copy(
                buf.at[pl.ds(0, half)],
                out_hbm.at[b].at[pl.ds(base, half)], sem_w))
        r1.wait()
        for b in range(B):
            writes.append(pltpu.async_copy(
                buf.at[pl.ds(half, half)],
                out_hbm.at[b].at[pl.ds(base + half, half)], sem_w))
        for w in writes:
            w.wait()

    return k(table)


def kernel(x, table):
    B = x.shape[0]
    return _broadcast_table(table, B)
